# baseline (speedup 1.0000x reference)
import functools

import jax
import jax.numpy as jnp
from jax import lax
from jax.experimental import pallas as pl
from jax.experimental.pallas import tpu as pltpu
from jax.experimental.pallas import tpu_sc as plsc

_B, _C, _J = 2, 12, 23
_H, _W = 256, 320
_HW = _H * _W
_NWORDS = _HW // 2
_G = 40
_G3 = _G ** 3
_HALF = _G // 2
_SPACING = 2.0
_OFFSET = -100.0
_NW = 32
_NTASK = _B * _J
_CHUNK = 8000
_NCHUNK = _G3 // _CHUNK
_ITERS = _CHUNK // 16


def _sc_gather_mean(hm_words, sub_idx):
    mesh = plsc.VectorSubcoreMesh(core_axis_name="c", subcore_axis_name="s")

    @functools.partial(
        pl.kernel,
        out_type=jax.ShapeDtypeStruct((_B * _J * _G3,), jnp.float32),
        mesh=mesh,
        compiler_params=pltpu.CompilerParams(needs_layout_passes=False),
        scratch_types=[
            pltpu.VMEM((_NWORDS,), jnp.int32),
            pltpu.VMEM((_G3,), jnp.float32),
            pltpu.VMEM((_CHUNK,), jnp.int32),
        ],
    )
    def run(hm_hbm, idx_hbm, out_hbm, plane_v, acc_v, idx_v):
        wid = lax.axis_index("s") * 2 + lax.axis_index("c")

        def task(t):
            b = t // _J
            j = t - b * _J

            @plsc.parallel_loop(0, _G3 // 16, unroll=4)
            def _zero(i):
                acc_v[pl.ds(i * 16, 16)] = jnp.zeros((16,), jnp.float32)

            def cam(c, carry):
                plane_base = ((b * _C + c) * _J + j) * _NWORDS
                pltpu.sync_copy(hm_hbm.at[pl.ds(plane_base, _NWORDS)], plane_v)

                def chunk(k, carry2):
                    idx_base = (b * _C + c) * _G3 + k * _CHUNK
                    pltpu.sync_copy(
                        idx_hbm.at[pl.ds(idx_base, _CHUNK)], idx_v)

                    @plsc.parallel_loop(0, _ITERS, unroll=4)
                    def _gather(i):
                        iv = idx_v[pl.ds(i * 16, 16)]
                        w = plsc.load_gather(plane_v, [iv >> 1])
                        hi = w & jnp.int32(-65536)
                        lo = w << 16
                        bits = jnp.where((iv & 1) == 1, hi, lo)
                        val = plsc.bitcast(bits, jnp.float32)
                        plsc.addupdate(
                            acc_v.at[pl.ds(k * _CHUNK + i * 16, 16)], val)

                    return carry2

                lax.fori_loop(0, _NCHUNK, chunk, 0)
                return carry

            lax.fori_loop(0, _C, cam, 0)

            @plsc.parallel_loop(0, _G3 // 16, unroll=4)
            def _scale(i):
                sl = pl.ds(i * 16, 16)
                acc_v[sl] = acc_v[sl] * jnp.float32(1.0 / _C)

            pltpu.sync_copy(acc_v, out_hbm.at[pl.ds((b * _J + j) * _G3, _G3)])

        task(wid)

        @pl.when(wid < _NTASK - _NW)
        def _second():
            task(wid + _NW)

    return run(hm_words, sub_idx)


def kernel(heatmaps, center, reproLookup):
    hm_bf = heatmaps.astype(jnp.bfloat16).reshape(_B * _C * _J * _NWORDS, 2)
    hm_words = lax.bitcast_convert_type(hm_bf, jnp.int32)
    cidx = ((center - _OFFSET) / _SPACING).astype(jnp.int32)
    starts = cidx - _HALF

    def slice_b(s):
        return lax.dynamic_slice(
            reproLookup, (jnp.int32(0), s[0], s[1], s[2]), (_C, _G, _G, _G))

    sub_idx = jax.vmap(slice_b)(starts).reshape(_B * _C * _G3)
    out = _sc_gather_mean(hm_words, sub_idx)
    return out.reshape(_B, _J, _G, _G, _G)


# speedup vs baseline: 12.9110x; 12.9110x over previous
import functools

import jax
import jax.numpy as jnp
from jax import lax
from jax.experimental import pallas as pl
from jax.experimental.pallas import tpu as pltpu
from jax.experimental.pallas import tpu_sc as plsc

_B, _C, _J = 2, 12, 23
_H, _W = 256, 320
_HW = _H * _W
_NWORDS = _HW // 2
_G = 40
_G3 = _G ** 3
_HALF = _G // 2
_SPACING = 2.0
_OFFSET = -100.0
_NW = 32
_NTASK = _B * _J
_CHUNK = 8000
_NCHUNK = _G3 // _CHUNK
_ITERS = _CHUNK // 16


def _sc_gather_mean(hm_words, sub_idx):
    mesh = plsc.VectorSubcoreMesh(core_axis_name="c", subcore_axis_name="s")

    @functools.partial(
        pl.kernel,
        out_type=jax.ShapeDtypeStruct((_B * _J * _G3,), jnp.float32),
        mesh=mesh,
        compiler_params=pltpu.CompilerParams(needs_layout_passes=False),
        scratch_types=[
            pltpu.VMEM((_NWORDS,), jnp.int32),
            pltpu.VMEM((_G3,), jnp.float32),
            pltpu.VMEM((_CHUNK,), jnp.int32),
        ],
    )
    def run(hm_hbm, idx_hbm, out_hbm, plane_v, acc_v, idx_v):
        wid = lax.axis_index("s") * 2 + lax.axis_index("c")

        def task(t):
            b = t // _J
            j = t - b * _J

            @plsc.parallel_loop(0, _G3 // 16, unroll=4)
            def _zero(i):
                acc_v[pl.ds(i * 16, 16)] = jnp.zeros((16,), jnp.float32)

            def cam(c, carry):
                plane_base = ((b * _C + c) * _J + j) * _NWORDS
                pltpu.sync_copy(hm_hbm.at[pl.ds(plane_base, _NWORDS)], plane_v)

                def chunk(k, carry2):
                    idx_base = (b * _C + c) * _G3 + k * _CHUNK
                    pltpu.sync_copy(
                        idx_hbm.at[pl.ds(idx_base, _CHUNK)], idx_v)

                    @plsc.parallel_loop(0, _ITERS, unroll=8)
                    def _gather(i):
                        iv = idx_v[pl.ds(i * 16, 16)]
                        in_hi = iv >= _NWORDS
                        wi = iv - jnp.where(in_hi, _NWORDS, 0)
                        w = plsc.load_gather(plane_v, [wi])
                        hi = w & jnp.int32(-65536)
                        lo = w << 16
                        bits = jnp.where(in_hi, hi, lo)
                        val = plsc.bitcast(bits, jnp.float32)
                        plsc.addupdate(
                            acc_v.at[pl.ds(k * _CHUNK + i * 16, 16)], val)

                    return carry2

                lax.fori_loop(0, _NCHUNK, chunk, 0)
                return carry

            lax.fori_loop(0, _C, cam, 0)

            @plsc.parallel_loop(0, _G3 // 16, unroll=4)
            def _scale(i):
                sl = pl.ds(i * 16, 16)
                acc_v[sl] = acc_v[sl] * jnp.float32(1.0 / _C)

            pltpu.sync_copy(acc_v, out_hbm.at[pl.ds((b * _J + j) * _G3, _G3)])

        task(wid)

        @pl.when(wid < _NTASK - _NW)
        def _second():
            task(wid + _NW)

    return run(hm_words, sub_idx)


def kernel(heatmaps, center, reproLookup):
    u = lax.bitcast_convert_type(heatmaps.reshape(_B, _C, _J, _HW), jnp.uint32)
    b16 = (u + jnp.uint32(0x7FFF) + ((u >> 16) & jnp.uint32(1))) >> 16
    wlo = b16[..., :_NWORDS]
    whi = b16[..., _NWORDS:]
    hm_words = lax.bitcast_convert_type(
        wlo | (whi << 16), jnp.int32).reshape(_B * _C * _J * _NWORDS)
    cidx = ((center - _OFFSET) / _SPACING).astype(jnp.int32)
    starts = cidx - _HALF

    def slice_b(s):
        return lax.dynamic_slice(
            reproLookup, (jnp.int32(0), s[0], s[1], s[2]), (_C, _G, _G, _G))

    sub_idx = jax.vmap(slice_b)(starts).reshape(_B * _C * _G3)
    out = _sc_gather_mean(hm_words, sub_idx)
    return out.reshape(_B, _J, _G, _G, _G)


# speedup vs baseline: 23.4797x; 1.8186x over previous
import functools

import jax
import jax.numpy as jnp
from jax import lax
from jax.experimental import pallas as pl
from jax.experimental.pallas import tpu as pltpu
from jax.experimental.pallas import tpu_sc as plsc

_B, _C, _J = 2, 12, 23
_H, _W = 256, 320
_HW = _H * _W
_NWORDS = _HW // 2
_G = 40
_G3 = _G ** 3
_HALF = _G // 2
_SPACING = 2.0
_OFFSET = -100.0
_NW = 32
_NTASK = _B * _J * 2
_TPTS = _G3 // 2
_CHUNK = 8000
_NCHUNK = _TPTS // _CHUNK
_NSTEP = _C * _NCHUNK
_ITERS = _CHUNK // 16


def _sc_gather_mean(hm_words, sub_idx):
    mesh = plsc.VectorSubcoreMesh(core_axis_name="c", subcore_axis_name="s")

    @functools.partial(
        pl.kernel,
        out_type=jax.ShapeDtypeStruct((_B * _J * _G3,), jnp.float32),
        mesh=mesh,
        compiler_params=pltpu.CompilerParams(needs_layout_passes=False),
        scratch_types=[
            pltpu.VMEM((_NWORDS,), jnp.int32),
            pltpu.VMEM((_NWORDS,), jnp.int32),
            pltpu.VMEM((_TPTS,), jnp.float32),
            pltpu.VMEM((_CHUNK,), jnp.int32),
            pltpu.VMEM((_CHUNK,), jnp.int32),
            pltpu.SemaphoreType.DMA,
            pltpu.SemaphoreType.DMA,
            pltpu.SemaphoreType.DMA,
            pltpu.SemaphoreType.DMA,
        ],
    )
    def run(hm_hbm, idx_hbm, out_hbm, plane_v0, plane_v1, acc_v,
            idx_v0, idx_v1, psem0, psem1, isem0, isem1):
        wid = lax.axis_index("s") * 2 + lax.axis_index("c")
        planes = (plane_v0, plane_v1)
        idxs = (idx_v0, idx_v1)
        psems = (psem0, psem1)
        isems = (isem0, isem1)

        def task(t):
            b = t // (_J * 2)
            rem = t - b * (_J * 2)
            j = rem // 2
            h = rem - j * 2

            def plane_copy(c):
                base = ((b * _C + c) * _J + j) * _NWORDS
                return pltpu.make_async_copy(
                    hm_hbm.at[pl.ds(base, _NWORDS)],
                    planes[c % 2], psems[c % 2])

            def idx_copy(s):
                c, k = divmod(s, _NCHUNK)
                base = (b * _C + c) * _G3 + h * _TPTS + k * _CHUNK
                return pltpu.make_async_copy(
                    idx_hbm.at[pl.ds(base, _CHUNK)],
                    idxs[s % 2], isems[s % 2])

            @plsc.parallel_loop(0, _TPTS // 16, unroll=4)
            def _zero(i):
                acc_v[pl.ds(i * 16, 16)] = jnp.zeros((16,), jnp.float32)

            plane_copy(0).start()
            idx_copy(0).start()
            for s in range(_NSTEP):
                c, k = divmod(s, _NCHUNK)
                if s + 1 < _NSTEP:
                    idx_copy(s + 1).start()
                if k == 0:
                    plane_copy(c).wait()
                    if c + 1 < _C:
                        plane_copy(c + 1).start()
                idx_copy(s).wait()
                pbuf = planes[c % 2]
                ibuf = idxs[s % 2]

                @plsc.parallel_loop(0, _ITERS, unroll=8)
                def _gather(i):
                    iv = ibuf[pl.ds(i * 16, 16)]
                    in_hi = iv >= _NWORDS
                    wi = iv - jnp.where(in_hi, _NWORDS, 0)
                    w = plsc.load_gather(pbuf, [wi])
                    hi = w & jnp.int32(-65536)
                    lo = w << 16
                    bits = jnp.where(in_hi, hi, lo)
                    val = plsc.bitcast(bits, jnp.float32)
                    plsc.addupdate(
                        acc_v.at[pl.ds(k * _CHUNK + i * 16, 16)], val)

            @plsc.parallel_loop(0, _TPTS // 16, unroll=4)
            def _scale(i):
                sl = pl.ds(i * 16, 16)
                acc_v[sl] = acc_v[sl] * jnp.float32(1.0 / _C)

            out_base = (b * _J + j) * _G3 + h * _TPTS
            pltpu.sync_copy(acc_v, out_hbm.at[pl.ds(out_base, _TPTS)])

        def rounds(r, carry):
            t = wid + r * _NW

            @pl.when(t < _NTASK)
            def _():
                task(t)

            return carry

        lax.fori_loop(0, 3, rounds, 0)

    return run(hm_words, sub_idx)


def kernel(heatmaps, center, reproLookup):
    u = lax.bitcast_convert_type(heatmaps, jnp.uint32)
    b16 = (u + jnp.uint32(0x7FFF) + ((u >> 16) & jnp.uint32(1))) >> 16
    wlo = b16[:, :, :, : _H // 2, :]
    whi = b16[:, :, :, _H // 2 :, :]
    hm_words = lax.bitcast_convert_type(
        wlo | (whi << 16), jnp.int32).reshape(_B * _C * _J * _NWORDS)

    cidx = ((center - _OFFSET) / _SPACING).astype(jnp.int32)
    starts = cidx - _HALF

    def slice_b(s):
        return lax.dynamic_slice(
            reproLookup, (jnp.int32(0), s[0], s[1], s[2]), (_C, _G, _G, _G))

    sub_idx = jax.vmap(slice_b)(starts).reshape(_B * _C * _G3)
    out = _sc_gather_mean(hm_words, sub_idx)
    return out.reshape(_B, _J, _G, _G, _G)


# speedup vs baseline: 23.8805x; 1.0171x over previous
import functools

import jax
import jax.numpy as jnp
from jax import lax
from jax.experimental import pallas as pl
from jax.experimental.pallas import tpu as pltpu
from jax.experimental.pallas import tpu_sc as plsc

_B, _C, _J = 2, 12, 23
_H, _W = 256, 320
_HW = _H * _W
_NWORDS = _HW // 2
_G = 40
_G3 = _G ** 3
_HALF = _G // 2
_SPACING = 2.0
_OFFSET = -100.0
_NW = 32
_NTASK = _B * _J * 2
_TPTS = _G3 // 2
_CHUNK = 8000
_NCHUNK = _TPTS // _CHUNK
_ITERS = _CHUNK // 16
_CSPLIT = 6


def _make_sc_call(cam_lo, ncams, has_init, do_scale):
    mesh = plsc.VectorSubcoreMesh(core_axis_name="c", subcore_axis_name="s")
    nstep = ncams * _NCHUNK

    @functools.partial(
        pl.kernel,
        out_type=jax.ShapeDtypeStruct((_B * _J * _G3,), jnp.float32),
        mesh=mesh,
        compiler_params=pltpu.CompilerParams(needs_layout_passes=False),
        scratch_types=[
            pltpu.VMEM((_NWORDS,), jnp.int32),
            pltpu.VMEM((_NWORDS,), jnp.int32),
            pltpu.VMEM((_TPTS,), jnp.float32),
            pltpu.VMEM((_CHUNK,), jnp.int32),
            pltpu.VMEM((_CHUNK,), jnp.int32),
            pltpu.SemaphoreType.DMA,
            pltpu.SemaphoreType.DMA,
            pltpu.SemaphoreType.DMA,
            pltpu.SemaphoreType.DMA,
        ],
    )
    def run(*args):
        if has_init:
            hm_hbm, idx_hbm, init_hbm, out_hbm = args[:4]
            rest = args[4:]
        else:
            hm_hbm, idx_hbm, out_hbm = args[:3]
            rest = args[3:]
        (plane_v0, plane_v1, acc_v, idx_v0, idx_v1,
         psem0, psem1, isem0, isem1) = rest
        wid = lax.axis_index("s") * 2 + lax.axis_index("c")
        planes = (plane_v0, plane_v1)
        idxs = (idx_v0, idx_v1)
        psems = (psem0, psem1)
        isems = (isem0, isem1)

        def task(t):
            b = t // (_J * 2)
            rem = t - b * (_J * 2)
            j = rem // 2
            h = rem - j * 2
            out_base = (b * _J + j) * _G3 + h * _TPTS

            def plane_copy(c):
                base = ((b * ncams + c) * _J + j) * _NWORDS
                return pltpu.make_async_copy(
                    hm_hbm.at[pl.ds(base, _NWORDS)],
                    planes[c % 2], psems[c % 2])

            def idx_copy(s):
                c, k = divmod(s, _NCHUNK)
                base = (b * _C + cam_lo + c) * _G3 + h * _TPTS + k * _CHUNK
                return pltpu.make_async_copy(
                    idx_hbm.at[pl.ds(base, _CHUNK)],
                    idxs[s % 2], isems[s % 2])

            plane_copy(0).start()
            idx_copy(0).start()
            if has_init:
                pltpu.sync_copy(init_hbm.at[pl.ds(out_base, _TPTS)], acc_v)
            else:
                @plsc.parallel_loop(0, _TPTS // 16, unroll=4)
                def _zero(i):
                    acc_v[pl.ds(i * 16, 16)] = jnp.zeros((16,), jnp.float32)

            for s in range(nstep):
                c, k = divmod(s, _NCHUNK)
                if s + 1 < nstep:
                    idx_copy(s + 1).start()
                if k == 0:
                    plane_copy(c).wait()
                    if c + 1 < ncams:
                        plane_copy(c + 1).start()
                idx_copy(s).wait()
                pbuf = planes[c % 2]
                ibuf = idxs[s % 2]

                @plsc.parallel_loop(0, _ITERS, unroll=8)
                def _gather(i):
                    iv = ibuf[pl.ds(i * 16, 16)]
                    in_hi = iv >= _NWORDS
                    wi = iv - jnp.where(in_hi, _NWORDS, 0)
                    w = plsc.load_gather(pbuf, [wi])
                    hi = w & jnp.int32(-65536)
                    lo = w << 16
                    bits = jnp.where(in_hi, hi, lo)
                    val = plsc.bitcast(bits, jnp.float32)
                    plsc.addupdate(
                        acc_v.at[pl.ds(k * _CHUNK + i * 16, 16)], val)

            if do_scale:
                @plsc.parallel_loop(0, _TPTS // 16, unroll=4)
                def _scale(i):
                    sl = pl.ds(i * 16, 16)
                    acc_v[sl] = acc_v[sl] * jnp.float32(1.0 / _C)

            pltpu.sync_copy(acc_v, out_hbm.at[pl.ds(out_base, _TPTS)])

        def rounds(r, carry):
            t = wid + r * _NW

            @pl.when(t < _NTASK)
            def _():
                task(t)

            return carry

        lax.fori_loop(0, 3, rounds, 0)

    return run


def _pack(hm):
    u = lax.bitcast_convert_type(hm, jnp.uint32)
    b16 = (u + jnp.uint32(0x7FFF) + ((u >> 16) & jnp.uint32(1))) >> 16
    wlo = b16[:, :, :, : _H // 2, :]
    whi = b16[:, :, :, _H // 2 :, :]
    return lax.bitcast_convert_type(wlo | (whi << 16), jnp.int32).reshape(-1)


def kernel(heatmaps, center, reproLookup):
    hm_words1 = _pack(heatmaps[:, :_CSPLIT])
    hm_words2 = _pack(heatmaps[:, _CSPLIT:])

    cidx = ((center - _OFFSET) / _SPACING).astype(jnp.int32)
    starts = cidx - _HALF

    def slice_b(s):
        return lax.dynamic_slice(
            reproLookup, (jnp.int32(0), s[0], s[1], s[2]), (_C, _G, _G, _G))

    sub_idx = jax.vmap(slice_b)(starts).reshape(_B * _C * _G3)
    part = _make_sc_call(0, _CSPLIT, False, False)(hm_words1, sub_idx)
    out = _make_sc_call(_CSPLIT, _C - _CSPLIT, True, True)(
        hm_words2, sub_idx, part)
    return out.reshape(_B, _J, _G, _G, _G)
